# Initial kernel scaffold; baseline (speedup 1.0000x reference)
import functools

import jax
import jax.numpy as jnp
from jax import lax
from jax.experimental import pallas as pl
from jax.experimental.pallas import tpu as pltpu
from jax.experimental.pallas import tpu_sc as plsc

NN = 10000
NE = 320000
D = 128
NG = 64

NC, NS = 2, 16
NW = NC * NS
EPT = NE // NW
CH = 80
NCHUNK = EPT // CH
RPT = NN // NS
BLK = 1000
NBLK = NN // BLK

_f32 = jnp.float32
_i32 = jnp.int32

_mesh = plsc.VectorSubcoreMesh(core_axis_name="c", subcore_axis_name="s")


@functools.partial(
    pl.kernel,
    out_type=jax.ShapeDtypeStruct((NC, NN, 16), _f32),
    mesh=_mesh,
    scratch_types=[
        pltpu.VMEM((CH,), _i32),
        pltpu.VMEM((CH, 16), _f32),
        pltpu.VMEM_SHARED((NN, 16), _f32),
        pltpu.SemaphoreType.DMA,
    ],
)
def _sc_degree(dst_hbm, zeros_hbm, ones_hbm, out_hbm, dst_v, ones_v, accum, sem):
    c = lax.axis_index("c")
    s = lax.axis_index("s")
    wid = c * NS + s
    pltpu.sync_copy(ones_hbm, ones_v)
    pltpu.sync_copy(zeros_hbm, accum.at[pl.ds(s * RPT, RPT)])
    plsc.subcore_barrier()

    def chunk(j, carry):
        base = wid * EPT + j * CH
        pltpu.sync_copy(dst_hbm.at[pl.ds(base, CH)], dst_v)
        pltpu.sync_copy(ones_v, accum.at[dst_v], add=True)
        return carry

    lax.fori_loop(0, NCHUNK, chunk, 0)
    plsc.subcore_barrier()
    pltpu.sync_copy(accum.at[pl.ds(s * RPT, RPT)],
                    out_hbm.at[c, pl.ds(s * RPT, RPT)])


@functools.partial(
    pl.kernel,
    out_type=jax.ShapeDtypeStruct((NC, NN, D), _f32),
    mesh=_mesh,
    scratch_types=[
        pltpu.VMEM((CH,), _i32),
        pltpu.VMEM((CH,), _i32),
        pltpu.VMEM((CH, D), _f32),
        pltpu.VMEM_SHARED((NN, D), _f32),
        pltpu.SemaphoreType.DMA,
    ],
)
def _sc_agg(src_hbm, dst_hbm, u_hbm, zeros_hbm, out_hbm,
            src_v, dst_v, rows_v, accum, sem):
    c = lax.axis_index("c")
    s = lax.axis_index("s")
    wid = c * NS + s
    pltpu.sync_copy(zeros_hbm, accum.at[pl.ds(s * RPT, RPT)])
    plsc.subcore_barrier()

    def chunk(j, carry):
        base = wid * EPT + j * CH
        pltpu.sync_copy(src_hbm.at[pl.ds(base, CH)], src_v)
        pltpu.sync_copy(dst_hbm.at[pl.ds(base, CH)], dst_v)
        pltpu.async_copy(u_hbm.at[src_v], rows_v, sem).wait()
        pltpu.sync_copy(rows_v, accum.at[dst_v], add=True)
        return carry

    lax.fori_loop(0, NCHUNK, chunk, 0)
    plsc.subcore_barrier()
    pltpu.sync_copy(accum.at[pl.ds(s * RPT, RPT)],
                    out_hbm.at[c, pl.ds(s * RPT, RPT)])


def _tc1_body(degp_ref, x_ref, w1_ref, u1_ref, dinv_ref):
    deg = degp_ref[0, :, 0:1] + degp_ref[1, :, 0:1] + 1.0
    dinv = lax.rsqrt(deg)
    h = jnp.dot(x_ref[...], w1_ref[...], preferred_element_type=_f32)
    u1_ref[...] = h * dinv
    dinv_ref[...] = jnp.broadcast_to(dinv, (BLK, 16))


_tc1 = pl.pallas_call(
    _tc1_body,
    grid=(NBLK,),
    in_specs=[
        pl.BlockSpec((NC, BLK, 16), lambda i: (0, i, 0)),
        pl.BlockSpec((BLK, D), lambda i: (i, 0)),
        pl.BlockSpec((D, D), lambda i: (0, 0)),
    ],
    out_specs=[
        pl.BlockSpec((BLK, D), lambda i: (i, 0)),
        pl.BlockSpec((BLK, 16), lambda i: (i, 0)),
    ],
    out_shape=[
        jax.ShapeDtypeStruct((NN, D), _f32),
        jax.ShapeDtypeStruct((NN, 16), _f32),
    ],
)


def _tc2_body(aggp_ref, u1_ref, dinv_ref, w2_ref, b1_ref, u2_ref):
    dinv = dinv_ref[:, 0:1]
    t = aggp_ref[0] + aggp_ref[1] + u1_ref[...]
    out1 = jnp.maximum(t * dinv + b1_ref[...], 0.0)
    h2 = jnp.dot(out1, w2_ref[...], preferred_element_type=_f32)
    u2_ref[...] = h2 * dinv


_tc2 = pl.pallas_call(
    _tc2_body,
    grid=(NBLK,),
    in_specs=[
        pl.BlockSpec((NC, BLK, D), lambda i: (0, i, 0)),
        pl.BlockSpec((BLK, D), lambda i: (i, 0)),
        pl.BlockSpec((BLK, 16), lambda i: (i, 0)),
        pl.BlockSpec((D, D), lambda i: (0, 0)),
        pl.BlockSpec((1, D), lambda i: (0, 0)),
    ],
    out_specs=pl.BlockSpec((BLK, D), lambda i: (i, 0)),
    out_shape=jax.ShapeDtypeStruct((NN, D), _f32),
)


def _tc3_body(aggp_ref, u2_ref, dinv_ref, b2_ref, batch_ref, out_ref,
              acc_ref, cnt_ref):
    i = pl.program_id(0)

    @pl.when(i == 0)
    def _():
        acc_ref[...] = jnp.zeros_like(acc_ref)
        cnt_ref[...] = jnp.zeros_like(cnt_ref)

    dinv = dinv_ref[:, 0:1]
    t = (aggp_ref[0] + aggp_ref[1] + u2_ref[...]) * dinv
    b = batch_ref[0]
    gids = lax.broadcasted_iota(_i32, (BLK, NG), 1)
    onehot = (b == gids).astype(_f32)
    acc_ref[...] += lax.dot_general(
        onehot, t, (((0,), (0,)), ((), ())), preferred_element_type=_f32)
    cnt_ref[...] += lax.dot_general(
        onehot, jnp.ones((BLK, D), _f32), (((0,), (0,)), ((), ())),
        preferred_element_type=_f32)

    @pl.when(i == pl.num_programs(0) - 1)
    def _():
        out_ref[...] = acc_ref[...] / jnp.maximum(cnt_ref[...], 1.0) + b2_ref[...]


_tc3 = pl.pallas_call(
    _tc3_body,
    grid=(NBLK,),
    in_specs=[
        pl.BlockSpec((NC, BLK, D), lambda i: (0, i, 0)),
        pl.BlockSpec((BLK, D), lambda i: (i, 0)),
        pl.BlockSpec((BLK, 16), lambda i: (i, 0)),
        pl.BlockSpec((1, D), lambda i: (0, 0)),
        pl.BlockSpec((1, BLK, 1), lambda i: (i, 0, 0)),
    ],
    out_specs=pl.BlockSpec((NG, D), lambda i: (0, 0)),
    out_shape=jax.ShapeDtypeStruct((NG, D), _f32),
    scratch_shapes=[
        pltpu.VMEM((NG, D), _f32),
        pltpu.VMEM((NG, D), _f32),
    ],
)


def kernel(x, edge_index, batch, W1, b1, W2, b2):
    src = edge_index[0].astype(_i32)
    dst = edge_index[1].astype(_i32)
    zdeg = jnp.zeros((RPT, 16), _f32)
    ones = jnp.ones((CH, 16), _f32)
    zrow = jnp.zeros((RPT, D), _f32)
    batch_r = batch.astype(_i32).reshape(NBLK, BLK, 1)

    degp = _sc_degree(dst, zdeg, ones)
    u1, dinv = _tc1(degp, x, W1)
    aggp1 = _sc_agg(src, dst, u1, zrow)
    u2 = _tc2(aggp1, u1, dinv, W2, b1.reshape(1, D))
    aggp2 = _sc_agg(src, dst, u2, zrow)
    out = _tc3(aggp2, u2, dinv, b2.reshape(1, D), batch_r)
    return out


# speedup vs baseline: 12.9418x; 12.9418x over previous
import functools

import jax
import jax.numpy as jnp
from jax import lax
from jax.experimental import pallas as pl
from jax.experimental.pallas import tpu as pltpu
from jax.experimental.pallas import tpu_sc as plsc

NN = 10000
NE = 320000
D = 128
NG = 64

NC, NS = 2, 16
NW = NC * NS
EPT = NE // NW
CH = 80
NCHUNK = EPT // CH
NNP = 10240
RPT = NNP // NS
BLK = 1000
NBLK = NN // BLK

_f32 = jnp.float32
_i32 = jnp.int32

_mesh = plsc.VectorSubcoreMesh(core_axis_name="c", subcore_axis_name="s")


@functools.partial(
    pl.kernel,
    out_type=jax.ShapeDtypeStruct((NC, NNP, 16), _f32),
    mesh=_mesh,
    scratch_types=[
        pltpu.VMEM((CH,), _i32),
        pltpu.VMEM((CH, 16), _f32),
        pltpu.VMEM_SHARED((NNP, 16), _f32),
        pltpu.SemaphoreType.DMA,
    ],
)
def _sc_degree(dst_hbm, zeros_hbm, ones_hbm, out_hbm, dst_v, ones_v, accum, sem):
    c = lax.axis_index("c")
    s = lax.axis_index("s")
    wid = c * NS + s
    pltpu.sync_copy(ones_hbm, ones_v)
    pltpu.sync_copy(zeros_hbm, accum.at[pl.ds(s * RPT, RPT)])
    plsc.subcore_barrier()

    def chunk(j, carry):
        base = wid * EPT + j * CH
        pltpu.sync_copy(dst_hbm.at[pl.ds(base, CH)], dst_v)
        pltpu.sync_copy(ones_v, accum.at[dst_v], add=True)
        return carry

    lax.fori_loop(0, NCHUNK, chunk, 0)
    plsc.subcore_barrier()
    pltpu.sync_copy(accum.at[pl.ds(s * RPT, RPT)],
                    out_hbm.at[c, pl.ds(s * RPT, RPT)])


@functools.partial(
    pl.kernel,
    out_type=jax.ShapeDtypeStruct((NC, NNP, D), _f32),
    mesh=_mesh,
    scratch_types=[
        pltpu.VMEM((CH,), _i32),
        pltpu.VMEM((CH,), _i32),
        pltpu.VMEM((CH, D), _f32),
        pltpu.VMEM_SHARED((NNP, D), _f32),
        pltpu.SemaphoreType.DMA,
    ],
)
def _sc_agg(src_hbm, dst_hbm, u_hbm, zeros_hbm, out_hbm,
            src_v, dst_v, rows_v, accum, sem):
    c = lax.axis_index("c")
    s = lax.axis_index("s")
    wid = c * NS + s
    pltpu.sync_copy(zeros_hbm, accum.at[pl.ds(s * RPT, RPT)])
    plsc.subcore_barrier()

    def chunk(j, carry):
        base = wid * EPT + j * CH
        pltpu.sync_copy(src_hbm.at[pl.ds(base, CH)], src_v)
        pltpu.sync_copy(dst_hbm.at[pl.ds(base, CH)], dst_v)
        pltpu.async_copy(u_hbm.at[src_v], rows_v, sem).wait()
        pltpu.sync_copy(rows_v, accum.at[dst_v], add=True)
        return carry

    lax.fori_loop(0, NCHUNK, chunk, 0)
    plsc.subcore_barrier()
    pltpu.sync_copy(accum.at[pl.ds(s * RPT, RPT)],
                    out_hbm.at[c, pl.ds(s * RPT, RPT)])


def _tc1_body(degp_ref, x_ref, w1_ref, u1_ref, dinv_ref):
    deg = degp_ref[0, :, 0:1] + degp_ref[1, :, 0:1] + 1.0
    dinv = lax.rsqrt(deg)
    h = jnp.dot(x_ref[...], w1_ref[...], preferred_element_type=_f32)
    u1_ref[...] = h * dinv
    dinv_ref[...] = jnp.broadcast_to(dinv, (BLK, 16))


_tc1 = pl.pallas_call(
    _tc1_body,
    grid=(NBLK,),
    in_specs=[
        pl.BlockSpec((NC, BLK, 16), lambda i: (0, i, 0)),
        pl.BlockSpec((BLK, D), lambda i: (i, 0)),
        pl.BlockSpec((D, D), lambda i: (0, 0)),
    ],
    out_specs=[
        pl.BlockSpec((BLK, D), lambda i: (i, 0)),
        pl.BlockSpec((BLK, 16), lambda i: (i, 0)),
    ],
    out_shape=[
        jax.ShapeDtypeStruct((NN, D), _f32),
        jax.ShapeDtypeStruct((NN, 16), _f32),
    ],
)


def _tc2_body(aggp_ref, u1_ref, dinv_ref, w2_ref, b1_ref, u2_ref):
    dinv = dinv_ref[:, 0:1]
    t = aggp_ref[0] + aggp_ref[1] + u1_ref[...]
    out1 = jnp.maximum(t * dinv + b1_ref[...], 0.0)
    h2 = jnp.dot(out1, w2_ref[...], preferred_element_type=_f32)
    u2_ref[...] = h2 * dinv


_tc2 = pl.pallas_call(
    _tc2_body,
    grid=(NBLK,),
    in_specs=[
        pl.BlockSpec((NC, BLK, D), lambda i: (0, i, 0)),
        pl.BlockSpec((BLK, D), lambda i: (i, 0)),
        pl.BlockSpec((BLK, 16), lambda i: (i, 0)),
        pl.BlockSpec((D, D), lambda i: (0, 0)),
        pl.BlockSpec((1, D), lambda i: (0, 0)),
    ],
    out_specs=pl.BlockSpec((BLK, D), lambda i: (i, 0)),
    out_shape=jax.ShapeDtypeStruct((NN, D), _f32),
)


def _tc3_body(aggp_ref, u2_ref, dinv_ref, b2_ref, batch_ref, out_ref,
              acc_ref, cnt_ref):
    i = pl.program_id(0)

    @pl.when(i == 0)
    def _():
        acc_ref[...] = jnp.zeros_like(acc_ref)
        cnt_ref[...] = jnp.zeros_like(cnt_ref)

    dinv = dinv_ref[:, 0:1]
    t = (aggp_ref[0] + aggp_ref[1] + u2_ref[...]) * dinv
    b = batch_ref[0]
    gids = lax.broadcasted_iota(_i32, (BLK, NG), 1)
    onehot = (b == gids).astype(_f32)
    acc_ref[...] += lax.dot_general(
        onehot, t, (((0,), (0,)), ((), ())), preferred_element_type=_f32)
    cnt_ref[...] += lax.dot_general(
        onehot, jnp.ones((BLK, D), _f32), (((0,), (0,)), ((), ())),
        preferred_element_type=_f32)

    @pl.when(i == pl.num_programs(0) - 1)
    def _():
        out_ref[...] = acc_ref[...] / jnp.maximum(cnt_ref[...], 1.0) + b2_ref[...]


_tc3 = pl.pallas_call(
    _tc3_body,
    grid=(NBLK,),
    in_specs=[
        pl.BlockSpec((NC, BLK, D), lambda i: (0, i, 0)),
        pl.BlockSpec((BLK, D), lambda i: (i, 0)),
        pl.BlockSpec((BLK, 16), lambda i: (i, 0)),
        pl.BlockSpec((1, D), lambda i: (0, 0)),
        pl.BlockSpec((1, BLK, 1), lambda i: (i, 0, 0)),
    ],
    out_specs=pl.BlockSpec((NG, D), lambda i: (0, 0)),
    out_shape=jax.ShapeDtypeStruct((NG, D), _f32),
    scratch_shapes=[
        pltpu.VMEM((NG, D), _f32),
        pltpu.VMEM((NG, D), _f32),
    ],
)


def kernel(x, edge_index, batch, W1, b1, W2, b2):
    src = edge_index[0].astype(_i32)
    dst = edge_index[1].astype(_i32)
    zdeg = jnp.zeros((RPT, 16), _f32)
    ones = jnp.ones((CH, 16), _f32)
    zrow = jnp.zeros((RPT, D), _f32)
    batch_r = batch.astype(_i32).reshape(NBLK, BLK, 1)

    degp = _sc_degree(dst, zdeg, ones)
    u1, dinv = _tc1(degp, x, W1)
    aggp1 = _sc_agg(src, dst, u1, zrow)
    u2 = _tc2(aggp1, u1, dinv, W2, b1.reshape(1, D))
    aggp2 = _sc_agg(src, dst, u2, zrow)
    out = _tc3(aggp2, u2, dinv, b2.reshape(1, D), batch_r)
    return out


# speedup vs baseline: 20.8272x; 1.6093x over previous
import functools

import jax
import jax.numpy as jnp
from jax import lax
from jax.experimental import pallas as pl
from jax.experimental.pallas import tpu as pltpu
from jax.experimental.pallas import tpu_sc as plsc

NN = 10000
NE = 320000
D = 128
NG = 64

NC, NS = 2, 16
NW = NC * NS
EPT = NE // NW
CH = 40
NCHUNK = EPT // CH
G = 5
GE = G * CH
NGRP = EPT // GE
NNP = 10240
RPT = NNP // NS
BLK = 1000
NBLK = NN // BLK

_f32 = jnp.float32
_i32 = jnp.int32

_mesh = plsc.VectorSubcoreMesh(core_axis_name="c", subcore_axis_name="s")


@functools.partial(
    pl.kernel,
    out_type=jax.ShapeDtypeStruct((NC, NNP, 16), _f32),
    mesh=_mesh,
    scratch_types=[
        [pltpu.VMEM((CH,), _i32) for _ in range(G)],
        pltpu.VMEM((CH, 16), _f32),
        pltpu.VMEM_SHARED((NNP, 16), _f32),
        pltpu.SemaphoreType.DMA,
        pltpu.SemaphoreType.DMA,
    ],
)
def _sc_degree(dst_hbm, zeros_hbm, ones_hbm, out_hbm, dst_bufs, ones_v,
               accum, isem, ssem):
    c = lax.axis_index("c")
    s = lax.axis_index("s")
    wid = c * NS + s
    pltpu.sync_copy(ones_hbm, ones_v)
    pltpu.sync_copy(zeros_hbm, accum.at[pl.ds(s * RPT, RPT)])
    plsc.subcore_barrier()

    def group(g, carry):
        base = wid * EPT + g * GE
        ic = [pltpu.async_copy(dst_hbm.at[pl.ds(base + b * CH, CH)],
                               dst_bufs[b], isem) for b in range(G)]
        for c_ in ic:
            c_.wait()
        sc = [pltpu.async_copy(ones_v, accum.at[dst_bufs[b]], ssem, add=True)
              for b in range(G)]
        for c_ in sc:
            c_.wait()
        return carry

    lax.fori_loop(0, NGRP, group, 0)
    plsc.subcore_barrier()
    pltpu.sync_copy(accum.at[pl.ds(s * RPT, RPT)],
                    out_hbm.at[c, pl.ds(s * RPT, RPT)])


@functools.partial(
    pl.kernel,
    out_type=jax.ShapeDtypeStruct((NC, NNP, D), _f32),
    mesh=_mesh,
    scratch_types=[
        pltpu.VMEM((GE,), _i32),
        [pltpu.VMEM((CH,), _i32) for _ in range(G)],
        pltpu.VMEM((G, CH, D), _f32),
        pltpu.VMEM_SHARED((NNP, D), _f32),
        pltpu.SemaphoreType.DMA,
        pltpu.SemaphoreType.DMA,
        pltpu.SemaphoreType.DMA,
    ],
)
def _sc_agg(src_hbm, dst_hbm, u_hbm, zeros_hbm, out_hbm,
            src_v, dst_bufs, rows_v, accum, isem, gsem, ssem):
    c = lax.axis_index("c")
    s = lax.axis_index("s")
    wid = c * NS + s
    pltpu.sync_copy(zeros_hbm, accum.at[pl.ds(s * RPT, RPT)])
    plsc.subcore_barrier()

    def group(g, carry):
        base = wid * EPT + g * GE
        ic = [pltpu.async_copy(src_hbm.at[pl.ds(base, GE)], src_v, isem)]
        ic += [pltpu.async_copy(dst_hbm.at[pl.ds(base + b * CH, CH)],
                                dst_bufs[b], isem) for b in range(G)]
        for c_ in ic:
            c_.wait()
        gc = [pltpu.async_copy(u_hbm.at[src_v.at[pl.ds(b * CH, CH)]],
                               rows_v.at[b], gsem) for b in range(G)]
        for c_ in gc:
            c_.wait()
        sc = [pltpu.async_copy(rows_v.at[b], accum.at[dst_bufs[b]], ssem,
                               add=True) for b in range(G)]
        for c_ in sc:
            c_.wait()
        return carry

    lax.fori_loop(0, NGRP, group, 0)
    plsc.subcore_barrier()
    pltpu.sync_copy(accum.at[pl.ds(s * RPT, RPT)],
                    out_hbm.at[c, pl.ds(s * RPT, RPT)])


def _tc1_body(degp_ref, x_ref, w1_ref, u1_ref, dinv_ref):
    deg = degp_ref[0, :, 0:1] + degp_ref[1, :, 0:1] + 1.0
    dinv = lax.rsqrt(deg)
    h = jnp.dot(x_ref[...], w1_ref[...], preferred_element_type=_f32)
    u1_ref[...] = h * dinv
    dinv_ref[...] = jnp.broadcast_to(dinv, (BLK, 16))


_tc1 = pl.pallas_call(
    _tc1_body,
    grid=(NBLK,),
    in_specs=[
        pl.BlockSpec((NC, BLK, 16), lambda i: (0, i, 0)),
        pl.BlockSpec((BLK, D), lambda i: (i, 0)),
        pl.BlockSpec((D, D), lambda i: (0, 0)),
    ],
    out_specs=[
        pl.BlockSpec((BLK, D), lambda i: (i, 0)),
        pl.BlockSpec((BLK, 16), lambda i: (i, 0)),
    ],
    out_shape=[
        jax.ShapeDtypeStruct((NN, D), _f32),
        jax.ShapeDtypeStruct((NN, 16), _f32),
    ],
)


def _tc2_body(aggp_ref, u1_ref, dinv_ref, w2_ref, b1_ref, u2_ref):
    dinv = dinv_ref[:, 0:1]
    t = aggp_ref[0] + aggp_ref[1] + u1_ref[...]
    out1 = jnp.maximum(t * dinv + b1_ref[...], 0.0)
    h2 = jnp.dot(out1, w2_ref[...], preferred_element_type=_f32)
    u2_ref[...] = h2 * dinv


_tc2 = pl.pallas_call(
    _tc2_body,
    grid=(NBLK,),
    in_specs=[
        pl.BlockSpec((NC, BLK, D), lambda i: (0, i, 0)),
        pl.BlockSpec((BLK, D), lambda i: (i, 0)),
        pl.BlockSpec((BLK, 16), lambda i: (i, 0)),
        pl.BlockSpec((D, D), lambda i: (0, 0)),
        pl.BlockSpec((1, D), lambda i: (0, 0)),
    ],
    out_specs=pl.BlockSpec((BLK, D), lambda i: (i, 0)),
    out_shape=jax.ShapeDtypeStruct((NN, D), _f32),
)


def _tc3_body(aggp_ref, u2_ref, dinv_ref, b2_ref, batch_ref, out_ref,
              acc_ref, cnt_ref):
    i = pl.program_id(0)

    @pl.when(i == 0)
    def _():
        acc_ref[...] = jnp.zeros_like(acc_ref)
        cnt_ref[...] = jnp.zeros_like(cnt_ref)

    dinv = dinv_ref[:, 0:1]
    t = (aggp_ref[0] + aggp_ref[1] + u2_ref[...]) * dinv
    b = batch_ref[0]
    gids = lax.broadcasted_iota(_i32, (BLK, NG), 1)
    onehot = (b == gids).astype(_f32)
    acc_ref[...] += lax.dot_general(
        onehot, t, (((0,), (0,)), ((), ())), preferred_element_type=_f32)
    cnt_ref[...] += lax.dot_general(
        onehot, jnp.ones((BLK, D), _f32), (((0,), (0,)), ((), ())),
        preferred_element_type=_f32)

    @pl.when(i == pl.num_programs(0) - 1)
    def _():
        out_ref[...] = acc_ref[...] / jnp.maximum(cnt_ref[...], 1.0) + b2_ref[...]


_tc3 = pl.pallas_call(
    _tc3_body,
    grid=(NBLK,),
    in_specs=[
        pl.BlockSpec((NC, BLK, D), lambda i: (0, i, 0)),
        pl.BlockSpec((BLK, D), lambda i: (i, 0)),
        pl.BlockSpec((BLK, 16), lambda i: (i, 0)),
        pl.BlockSpec((1, D), lambda i: (0, 0)),
        pl.BlockSpec((1, BLK, 1), lambda i: (i, 0, 0)),
    ],
    out_specs=pl.BlockSpec((NG, D), lambda i: (0, 0)),
    out_shape=jax.ShapeDtypeStruct((NG, D), _f32),
    scratch_shapes=[
        pltpu.VMEM((NG, D), _f32),
        pltpu.VMEM((NG, D), _f32),
    ],
)


def kernel(x, edge_index, batch, W1, b1, W2, b2):
    src = edge_index[0].astype(_i32)
    dst = edge_index[1].astype(_i32)
    zdeg = jnp.zeros((RPT, 16), _f32)
    ones = jnp.ones((CH, 16), _f32)
    zrow = jnp.zeros((RPT, D), _f32)
    batch_r = batch.astype(_i32).reshape(NBLK, BLK, 1)

    degp = _sc_degree(dst, zdeg, ones)
    u1, dinv = _tc1(degp, x, W1)
    aggp1 = _sc_agg(src, dst, u1, zrow)
    u2 = _tc2(aggp1, u1, dinv, W2, b1.reshape(1, D))
    aggp2 = _sc_agg(src, dst, u2, zrow)
    out = _tc3(aggp2, u2, dinv, b2.reshape(1, D), batch_r)
    return out
